# initial kernel scaffold (unmeasured)
import jax
import jax.numpy as jnp
from jax import lax
from jax.experimental import pallas as pl
from jax.experimental.pallas import tpu as pltpu

T = 2048
V_HALF = 8192
V = 2 * V_HALF


def _stats_exchange(stats):

    def body(stats_ref, out_ref, send_sem, recv_sem):
        my_x = lax.axis_index("x")
        my_y = lax.axis_index("y")
        nbr = (1 - my_x, my_y)

        barrier = pltpu.get_barrier_semaphore()
        pl.semaphore_signal(
            barrier, inc=1, device_id=nbr, device_id_type=pl.DeviceIdType.MESH
        )
        pl.semaphore_wait(barrier, 1)

        rdma = pltpu.make_async_remote_copy(
            src_ref=stats_ref,
            dst_ref=out_ref,
            send_sem=send_sem,
            recv_sem=recv_sem,
            device_id=nbr,
            device_id_type=pl.DeviceIdType.MESH,
        )
        rdma.start()
        rdma.wait()

    return pl.pallas_call(
        body,
        out_shape=jax.ShapeDtypeStruct(stats.shape, stats.dtype),
        in_specs=[pl.BlockSpec(memory_space=pltpu.VMEM)],
        out_specs=pl.BlockSpec(memory_space=pltpu.VMEM),
        scratch_shapes=[pltpu.SemaphoreType.DMA, pltpu.SemaphoreType.DMA],
        compiler_params=pltpu.CompilerParams(collective_id=0),
    )(stats)


def _half_exchange(half):

    def body(half_ref, out_ref, copy_sem, send_sem, recv_sem):
        my_x = lax.axis_index("x")
        my_y = lax.axis_index("y")
        nbr = (1 - my_x, my_y)

        barrier = pltpu.get_barrier_semaphore()
        pl.semaphore_signal(
            barrier, inc=1, device_id=nbr, device_id_type=pl.DeviceIdType.MESH
        )
        pl.semaphore_wait(barrier, 1)

        local = pltpu.make_async_copy(
            half_ref,
            out_ref.at[:, pl.ds(my_x * V_HALF, V_HALF)],
            copy_sem,
        )
        local.start()

        rdma = pltpu.make_async_remote_copy(
            src_ref=half_ref,
            dst_ref=out_ref.at[:, pl.ds(my_x * V_HALF, V_HALF)],
            send_sem=send_sem,
            recv_sem=recv_sem,
            device_id=nbr,
            device_id_type=pl.DeviceIdType.MESH,
        )
        rdma.start()
        local.wait()
        rdma.wait()

    return pl.pallas_call(
        body,
        out_shape=jax.ShapeDtypeStruct((T, V), half.dtype),
        in_specs=[pl.BlockSpec(memory_space=pltpu.ANY)],
        out_specs=pl.BlockSpec(memory_space=pltpu.ANY),
        scratch_shapes=[
            pltpu.SemaphoreType.DMA,
            pltpu.SemaphoreType.DMA,
            pltpu.SemaphoreType.DMA,
        ],
        compiler_params=pltpu.CompilerParams(collective_id=1),
    )(half)


def kernel(x, W):
    logits = jnp.dot(
        x.astype(jnp.bfloat16),
        W.astype(jnp.bfloat16),
        preferred_element_type=jnp.float32,
    )

    m_loc = jnp.max(logits, axis=-1, keepdims=True)
    e_loc = jnp.exp(logits - m_loc)
    s_loc = jnp.sum(e_loc, axis=-1, keepdims=True)

    stats = jnp.concatenate([m_loc, s_loc], axis=1)
    rem = _stats_exchange(stats)
    m_rem = rem[:, 0:1]
    s_rem = rem[:, 1:2]

    m = jnp.maximum(m_loc, m_rem)
    s = s_loc * jnp.exp(m_loc - m) + s_rem * jnp.exp(m_rem - m)
    half = e_loc * (jnp.exp(m_loc - m) / s)

    return _half_exchange(half)


# baseline (device time: 2454590 ns/iter reference)
import jax
import jax.numpy as jnp
from jax import lax
from jax.experimental import pallas as pl
from jax.experimental.pallas import tpu as pltpu

T = 2048
V_HALF = 8192
V = 2 * V_HALF


def _stats_exchange(stats):

    def body(stats_ref, out_ref, send_sem, recv_sem):
        my_x = lax.axis_index("x")
        my_y = lax.axis_index("y")
        nbr = (1 - my_x, my_y)

        barrier = pltpu.get_barrier_semaphore()
        pl.semaphore_signal(
            barrier, inc=1, device_id=nbr, device_id_type=pl.DeviceIdType.MESH
        )
        pl.semaphore_wait(barrier, 1)

        rdma = pltpu.make_async_remote_copy(
            src_ref=stats_ref,
            dst_ref=out_ref,
            send_sem=send_sem,
            recv_sem=recv_sem,
            device_id=nbr,
            device_id_type=pl.DeviceIdType.MESH,
        )
        rdma.start()
        rdma.wait()

    return pl.pallas_call(
        body,
        out_shape=jax.ShapeDtypeStruct(stats.shape, stats.dtype),
        in_specs=[pl.BlockSpec(memory_space=pltpu.VMEM)],
        out_specs=pl.BlockSpec(memory_space=pltpu.VMEM),
        scratch_shapes=[pltpu.SemaphoreType.DMA, pltpu.SemaphoreType.DMA],
        compiler_params=pltpu.CompilerParams(collective_id=0),
    )(stats)


def _half_exchange(half):

    def body(half_ref, out_ref, copy_sem, send_sem, recv_sem):
        my_x = lax.axis_index("x")
        my_y = lax.axis_index("y")
        nbr = (1 - my_x, my_y)

        barrier = pltpu.get_barrier_semaphore()
        pl.semaphore_signal(
            barrier, inc=1, device_id=nbr, device_id_type=pl.DeviceIdType.MESH
        )
        pl.semaphore_wait(barrier, 1)

        local = pltpu.make_async_copy(
            half_ref,
            out_ref.at[:, pl.ds(my_x * V_HALF, V_HALF)],
            copy_sem,
        )
        local.start()

        rdma = pltpu.make_async_remote_copy(
            src_ref=half_ref,
            dst_ref=out_ref.at[:, pl.ds(my_x * V_HALF, V_HALF)],
            send_sem=send_sem,
            recv_sem=recv_sem,
            device_id=nbr,
            device_id_type=pl.DeviceIdType.MESH,
        )
        rdma.start()
        local.wait()
        rdma.wait()

    return pl.pallas_call(
        body,
        out_shape=jax.ShapeDtypeStruct((T, V), half.dtype),
        in_specs=[pl.BlockSpec(memory_space=pltpu.MemorySpace.HBM)],
        out_specs=pl.BlockSpec(memory_space=pltpu.MemorySpace.HBM),
        scratch_shapes=[
            pltpu.SemaphoreType.DMA,
            pltpu.SemaphoreType.DMA,
            pltpu.SemaphoreType.DMA,
        ],
        compiler_params=pltpu.CompilerParams(collective_id=1),
    )(half)


def kernel(x, W):
    logits = jnp.dot(
        x.astype(jnp.bfloat16),
        W.astype(jnp.bfloat16),
        preferred_element_type=jnp.float32,
    )

    m_loc = jnp.max(logits, axis=-1, keepdims=True)
    e_loc = jnp.exp(logits - m_loc)
    s_loc = jnp.sum(e_loc, axis=-1, keepdims=True)

    stats = jnp.concatenate([m_loc, s_loc], axis=1)
    rem = _stats_exchange(stats)
    m_rem = rem[:, 0:1]
    s_rem = rem[:, 1:2]

    m = jnp.maximum(m_loc, m_rem)
    s = s_loc * jnp.exp(m_loc - m) + s_rem * jnp.exp(m_rem - m)
    half = e_loc * (jnp.exp(m_loc - m) / s)

    return _half_exchange(half)


# device time: 904224 ns/iter; 2.7146x vs baseline; 2.7146x over previous
import jax
import jax.numpy as jnp
from jax import lax
from jax.experimental import pallas as pl
from jax.experimental.pallas import tpu as pltpu

T = 2048
V_HALF = 8192
V = 2 * V_HALF


def _stats_exchange(stats):

    def body(stats_ref, out_ref, send_sem, recv_sem):
        my_x = lax.axis_index("x")
        my_y = lax.axis_index("y")
        nbr = (1 - my_x, my_y)

        barrier = pltpu.get_barrier_semaphore()
        pl.semaphore_signal(
            barrier, inc=1, device_id=nbr, device_id_type=pl.DeviceIdType.MESH
        )
        pl.semaphore_wait(barrier, 1)

        rdma = pltpu.make_async_remote_copy(
            src_ref=stats_ref,
            dst_ref=out_ref,
            send_sem=send_sem,
            recv_sem=recv_sem,
            device_id=nbr,
            device_id_type=pl.DeviceIdType.MESH,
        )
        rdma.start()
        rdma.wait()

    return pl.pallas_call(
        body,
        out_shape=jax.ShapeDtypeStruct(stats.shape, stats.dtype),
        in_specs=[pl.BlockSpec(memory_space=pltpu.VMEM)],
        out_specs=pl.BlockSpec(memory_space=pltpu.VMEM),
        scratch_shapes=[pltpu.SemaphoreType.DMA, pltpu.SemaphoreType.DMA],
        compiler_params=pltpu.CompilerParams(collective_id=0),
    )(stats)


def _half_exchange(half):

    def body(half_ref, out_ref, send_sem, recv_sem):
        my_x = lax.axis_index("x")
        my_y = lax.axis_index("y")
        nbr = (1 - my_x, my_y)

        barrier = pltpu.get_barrier_semaphore()
        pl.semaphore_signal(
            barrier, inc=1, device_id=nbr, device_id_type=pl.DeviceIdType.MESH
        )
        pl.semaphore_wait(barrier, 1)

        rdma = pltpu.make_async_remote_copy(
            src_ref=half_ref,
            dst_ref=out_ref,
            send_sem=send_sem,
            recv_sem=recv_sem,
            device_id=nbr,
            device_id_type=pl.DeviceIdType.MESH,
        )
        rdma.start()
        rdma.wait()

    return pl.pallas_call(
        body,
        out_shape=jax.ShapeDtypeStruct((T, V_HALF), half.dtype),
        in_specs=[pl.BlockSpec(memory_space=pltpu.MemorySpace.HBM)],
        out_specs=pl.BlockSpec(memory_space=pltpu.MemorySpace.HBM),
        scratch_shapes=[
            pltpu.SemaphoreType.DMA,
            pltpu.SemaphoreType.DMA,
        ],
        compiler_params=pltpu.CompilerParams(collective_id=1),
    )(half)


def kernel(x, W):
    logits = jnp.dot(
        x.astype(jnp.bfloat16),
        W.astype(jnp.bfloat16),
        preferred_element_type=jnp.float32,
    )

    m_loc = jnp.max(logits, axis=-1, keepdims=True)
    e_loc = jnp.exp(logits - m_loc)
    s_loc = jnp.sum(e_loc, axis=-1, keepdims=True)

    stats = jnp.concatenate([m_loc, s_loc], axis=1)
    rem = _stats_exchange(stats)
    m_rem = rem[:, 0:1]
    s_rem = rem[:, 1:2]

    m = jnp.maximum(m_loc, m_rem)
    s = s_loc * jnp.exp(m_loc - m) + s_rem * jnp.exp(m_rem - m)
    half = e_loc * (jnp.exp(m_loc - m) / s)

    theirs = _half_exchange(half.astype(jnp.bfloat16)).astype(jnp.float32)

    my_x = lax.axis_index("x")
    return lax.cond(
        my_x == 0,
        lambda a, b: jnp.concatenate([a, b], axis=1),
        lambda a, b: jnp.concatenate([b, a], axis=1),
        half,
        theirs,
    )


# device time: 772861 ns/iter; 3.1760x vs baseline; 1.1700x over previous
import jax
import jax.numpy as jnp
from jax import lax
from jax.experimental import pallas as pl
from jax.experimental.pallas import tpu as pltpu

T = 2048
V_HALF = 8192
V = 2 * V_HALF
RC = 128
NC = T // RC


def _fused_softmax_exchange(logits):
    def body(
        logits_ref,
        out_ref,
        sbuf,
        rbuf,
        lbuf,
        ebuf,
        obuf,
        rconv,
        m_loc, s_loc, m_rem, s_rem, scale_own, scale_rem,
        lload_sems, estage_sems, ostore_sems, rload_sems,
        send_sems, recv_sems,
        stat_send_sems, stat_recv_sems,
    ):
        my_x = lax.axis_index("x")
        my_y = lax.axis_index("y")
        nbr = (1 - my_x, my_y)

        barrier = pltpu.get_barrier_semaphore()
        pl.semaphore_signal(
            barrier, inc=1, device_id=nbr, device_id_type=pl.DeviceIdType.MESH
        )
        pl.semaphore_wait(barrier, 1)

        def rows(k):
            return pl.ds(k * RC, RC)

        def load_logits(k, slot):
            return pltpu.make_async_copy(
                logits_ref.at[rows(k), :], lbuf.at[slot], lload_sems.at[slot]
            )

        def chunk_rdma(k):
            return pltpu.make_async_remote_copy(
                src_ref=sbuf.at[rows(k), :],
                dst_ref=rbuf.at[rows(k), :],
                send_sem=send_sems.at[k],
                recv_sem=recv_sems.at[k],
                device_id=nbr,
                device_id_type=pl.DeviceIdType.MESH,
            )

        load_logits(0, 0).start()
        for k in range(NC):
            slot = k % 2
            load_logits(k, slot).wait()
            if k + 1 < NC:
                load_logits(k + 1, (k + 1) % 2).start()
            l = lbuf[slot]
            mk = jnp.max(l, axis=1, keepdims=True)
            e = jnp.exp(l - mk)
            s_loc[rows(k), :] = jnp.sum(e, axis=1, keepdims=True)
            m_loc[rows(k), :] = mk
            ebuf[slot] = e.astype(jnp.bfloat16)
            stage = pltpu.make_async_copy(
                ebuf.at[slot], sbuf.at[rows(k), :], estage_sems.at[slot]
            )
            stage.start()
            stage.wait()
            chunk_rdma(k).start()

        st_m = pltpu.make_async_remote_copy(
            src_ref=m_loc,
            dst_ref=m_rem,
            send_sem=stat_send_sems.at[0],
            recv_sem=stat_recv_sems.at[0],
            device_id=nbr,
            device_id_type=pl.DeviceIdType.MESH,
        )
        st_s = pltpu.make_async_remote_copy(
            src_ref=s_loc,
            dst_ref=s_rem,
            send_sem=stat_send_sems.at[1],
            recv_sem=stat_recv_sems.at[1],
            device_id=nbr,
            device_id_type=pl.DeviceIdType.MESH,
        )
        st_m.start()
        st_s.start()
        st_m.wait()
        st_s.wait()

        mm = jnp.maximum(m_loc[...], m_rem[...])
        e_own = jnp.exp(m_loc[...] - mm)
        e_rem = jnp.exp(m_rem[...] - mm)
        ss = s_loc[...] * e_own + s_rem[...] * e_rem
        scale_own[...] = e_own / ss
        scale_rem[...] = e_rem / ss

        pending = {}

        def emit_half(src_hbm, scale_ref, col0, wait_recv_first):
            for k in range(NC):
                slot = k % 2
                if wait_recv_first:
                    chunk_rdma(k).wait_recv()
                ld = pltpu.make_async_copy(
                    src_hbm.at[rows(k), :], rconv.at[slot], rload_sems.at[slot]
                )
                ld.start()
                ld.wait()
                if slot in pending:
                    pending[slot].wait()
                obuf[slot] = rconv[slot].astype(jnp.float32) * scale_ref[rows(k), :]
                st = pltpu.make_async_copy(
                    obuf.at[slot],
                    out_ref.at[rows(k), pl.ds(col0, V_HALF)],
                    ostore_sems.at[slot],
                )
                st.start()
                pending[slot] = st

        emit_half(sbuf, scale_own, my_x * V_HALF, wait_recv_first=False)
        emit_half(rbuf, scale_rem, (1 - my_x) * V_HALF, wait_recv_first=True)
        for st in pending.values():
            st.wait()
        for k in range(NC):
            chunk_rdma(k).wait_send()

    out = pl.pallas_call(
        body,
        out_shape=[
            jax.ShapeDtypeStruct((T, V), jnp.float32),
            jax.ShapeDtypeStruct((T, V_HALF), jnp.bfloat16),
            jax.ShapeDtypeStruct((T, V_HALF), jnp.bfloat16),
        ],
        in_specs=[pl.BlockSpec(memory_space=pltpu.MemorySpace.HBM)],
        out_specs=[
            pl.BlockSpec(memory_space=pltpu.MemorySpace.HBM),
            pl.BlockSpec(memory_space=pltpu.MemorySpace.HBM),
            pl.BlockSpec(memory_space=pltpu.MemorySpace.HBM),
        ],
        scratch_shapes=[
            pltpu.VMEM((2, RC, V_HALF), jnp.float32),
            pltpu.VMEM((2, RC, V_HALF), jnp.bfloat16),
            pltpu.VMEM((2, RC, V_HALF), jnp.float32),
            pltpu.VMEM((2, RC, V_HALF), jnp.bfloat16),
            pltpu.VMEM((T, 1), jnp.float32),
            pltpu.VMEM((T, 1), jnp.float32),
            pltpu.VMEM((T, 1), jnp.float32),
            pltpu.VMEM((T, 1), jnp.float32),
            pltpu.VMEM((T, 1), jnp.float32),
            pltpu.VMEM((T, 1), jnp.float32),
            pltpu.SemaphoreType.DMA((2,)),
            pltpu.SemaphoreType.DMA((2,)),
            pltpu.SemaphoreType.DMA((2,)),
            pltpu.SemaphoreType.DMA((2,)),
            pltpu.SemaphoreType.DMA((NC,)),
            pltpu.SemaphoreType.DMA((NC,)),
            pltpu.SemaphoreType.DMA((2,)),
            pltpu.SemaphoreType.DMA((2,)),
        ],
        compiler_params=pltpu.CompilerParams(
            collective_id=0, vmem_limit_bytes=50 * 1024 * 1024
        ),
    )(logits)
    return out[0]


def kernel(x, W):
    logits = jnp.dot(
        x.astype(jnp.bfloat16),
        W.astype(jnp.bfloat16),
        preferred_element_type=jnp.float32,
    )
    return _fused_softmax_exchange(logits)


# device time: 625263 ns/iter; 3.9257x vs baseline; 1.2361x over previous
import jax
import jax.numpy as jnp
from jax import lax
from jax.experimental import pallas as pl
from jax.experimental.pallas import tpu as pltpu

T = 2048
D = 4096
V_HALF = 8192
V = 2 * V_HALF
CB = 512
NCB = V_HALF // CB
assert NCB <= 16

ML = 16
SL = 17


def _fused(x_bf16, W):
    def body(
        x_ref,
        W_ref,
        out_ref,
        sbuf,
        rbuf,
        wstage,
        wbuf,
        ebuf,
        rconv,
        obuf,
        stats_mine,
        stats_theirs,
        scales,
        mrun, srun,
        wload_sem, estage_sem, ostore_sem, rload_sem,
        stat_send_sem, stat_recv_sem,
        send_sems, recv_sems,
    ):
        my_x = lax.axis_index("x")
        my_y = lax.axis_index("y")
        nbr = (1 - my_x, my_y)

        barrier = pltpu.get_barrier_semaphore()
        pl.semaphore_signal(
            barrier, inc=1, device_id=nbr, device_id_type=pl.DeviceIdType.MESH
        )
        pl.semaphore_wait(barrier, 1)

        def wload(cb):
            return pltpu.make_async_copy(
                W_ref.at[:, pl.ds(cb * CB, CB)], wstage, wload_sem
            )

        def chunk_rdma(cb):
            return pltpu.make_async_remote_copy(
                src_ref=sbuf.at[cb],
                dst_ref=rbuf.at[cb],
                send_sem=send_sems.at[cb],
                recv_sem=recv_sems.at[cb],
                device_id=nbr,
                device_id_type=pl.DeviceIdType.MESH,
            )

        mrun[...] = jnp.full((T, 1), -jnp.inf, jnp.float32)
        srun[...] = jnp.zeros((T, 1), jnp.float32)
        wload(0).start()

        def phase1(cb, carry):
            wload(cb).wait()
            wbuf[...] = wstage[...].astype(jnp.bfloat16)

            @pl.when(cb + 1 < NCB)
            def _():
                wload(cb + 1).start()

            logits = jnp.dot(
                x_ref[...], wbuf[...], preferred_element_type=jnp.float32
            )
            m_cb = jnp.max(logits, axis=1, keepdims=True)
            e = jnp.exp(logits - m_cb)
            s_cb = jnp.sum(e, axis=1, keepdims=True)
            lane = lax.broadcasted_iota(jnp.int32, (T, 32), 1)
            stats_mine[...] = jnp.where(lane == cb, m_cb, stats_mine[...])
            m_new = jnp.maximum(mrun[...], m_cb)
            srun[...] = srun[...] * jnp.exp(mrun[...] - m_new) + s_cb * jnp.exp(
                m_cb - m_new
            )
            mrun[...] = m_new
            ebuf[...] = e.astype(jnp.bfloat16)
            stage = pltpu.make_async_copy(ebuf, sbuf.at[cb], estage_sem)
            stage.start()
            stage.wait()
            chunk_rdma(cb).start()
            return carry

        lax.fori_loop(0, NCB, phase1, 0)

        stats_mine[:, ML : ML + 1] = mrun[...]
        stats_mine[:, SL : SL + 1] = srun[...]
        st = pltpu.make_async_remote_copy(
            src_ref=stats_mine,
            dst_ref=stats_theirs,
            send_sem=stat_send_sem,
            recv_sem=stat_recv_sem,
            device_id=nbr,
            device_id_type=pl.DeviceIdType.MESH,
        )
        st.start()
        st.wait()

        m_loc = stats_mine[:, ML : ML + 1]
        s_loc = stats_mine[:, SL : SL + 1]
        m_rem = stats_theirs[:, ML : ML + 1]
        s_rem = stats_theirs[:, SL : SL + 1]
        mm = jnp.maximum(m_loc, m_rem)
        ss = s_loc * jnp.exp(m_loc - mm) + s_rem * jnp.exp(m_rem - mm)
        scales[:, 0:NCB] = jnp.exp(stats_mine[:, 0:NCB] - mm) / ss
        scales[:, 16 : 16 + NCB] = jnp.exp(stats_theirs[:, 0:NCB] - mm) / ss

        def make_emit(src_hbm, lane0, col0, wait_recv_first):
            def emit(cb, carry):
                if wait_recv_first:
                    chunk_rdma(cb).wait_recv()
                ld = pltpu.make_async_copy(src_hbm.at[cb], rconv, rload_sem)
                ld.start()
                ld.wait()
                lane = lax.broadcasted_iota(jnp.int32, (T, 32), 1)
                svec = jnp.sum(
                    jnp.where(lane == lane0 + cb, scales[...], 0.0),
                    axis=1,
                    keepdims=True,
                )
                obuf[...] = rconv[...].astype(jnp.float32) * svec
                stc = pltpu.make_async_copy(
                    obuf,
                    out_ref.at[:, pl.ds(col0 + cb * CB, CB)],
                    ostore_sem,
                )
                stc.start()
                stc.wait()
                return carry

            return emit

        lax.fori_loop(0, NCB, make_emit(sbuf, 0, my_x * V_HALF, False), 0)
        lax.fori_loop(0, NCB, make_emit(rbuf, 16, (1 - my_x) * V_HALF, True), 0)

        def waitsend(cb, carry):
            chunk_rdma(cb).wait_send()
            return carry

        lax.fori_loop(0, NCB, waitsend, 0)

    out = pl.pallas_call(
        body,
        out_shape=[
            jax.ShapeDtypeStruct((T, V), jnp.float32),
            jax.ShapeDtypeStruct((NCB, T, CB), jnp.bfloat16),
            jax.ShapeDtypeStruct((NCB, T, CB), jnp.bfloat16),
        ],
        in_specs=[
            pl.BlockSpec(memory_space=pltpu.MemorySpace.VMEM),
            pl.BlockSpec(memory_space=pltpu.MemorySpace.HBM),
        ],
        out_specs=[
            pl.BlockSpec(memory_space=pltpu.MemorySpace.HBM),
            pl.BlockSpec(memory_space=pltpu.MemorySpace.HBM),
            pl.BlockSpec(memory_space=pltpu.MemorySpace.HBM),
        ],
        scratch_shapes=[
            pltpu.VMEM((D, CB), jnp.float32),
            pltpu.VMEM((D, CB), jnp.bfloat16),
            pltpu.VMEM((T, CB), jnp.bfloat16),
            pltpu.VMEM((T, CB), jnp.bfloat16),
            pltpu.VMEM((T, CB), jnp.float32),
            pltpu.VMEM((T, 32), jnp.float32),
            pltpu.VMEM((T, 32), jnp.float32),
            pltpu.VMEM((T, 32), jnp.float32),
            pltpu.VMEM((T, 1), jnp.float32),
            pltpu.VMEM((T, 1), jnp.float32),
            pltpu.SemaphoreType.DMA,
            pltpu.SemaphoreType.DMA,
            pltpu.SemaphoreType.DMA,
            pltpu.SemaphoreType.DMA,
            pltpu.SemaphoreType.DMA,
            pltpu.SemaphoreType.DMA,
            pltpu.SemaphoreType.DMA((NCB,)),
            pltpu.SemaphoreType.DMA((NCB,)),
        ],
        compiler_params=pltpu.CompilerParams(
            collective_id=0, vmem_limit_bytes=56 * 1024 * 1024
        ),
    )(x_bf16, W)
    return out[0]


def kernel(x, W):
    return _fused(x.astype(jnp.bfloat16), W)


# device time: 550389 ns/iter; 4.4597x vs baseline; 1.1360x over previous
import jax
import jax.numpy as jnp
from jax import lax
from jax.experimental import pallas as pl
from jax.experimental.pallas import tpu as pltpu

T = 2048
D = 4096
V_HALF = 8192
V = 2 * V_HALF
CB = 512
NCB = V_HALF // CB
assert NCB <= 16

ML = 16
SL = 17


def _fused(x_bf16, W):
    def body(
        x_ref,
        W_ref,
        out_ref,
        sbuf,
        rbuf,
        wstage,
        wbuf,
        ebuf,
        rconv,
        obuf,
        stats_mine,
        stats_theirs,
        scales,
        mrun, srun,
        wload_sem, estage_sem, rload_sems, ostore_sems,
        stat_send_sem, stat_recv_sem,
        send_sems, recv_sems,
    ):
        my_x = lax.axis_index("x")
        my_y = lax.axis_index("y")
        nbr = (1 - my_x, my_y)

        barrier = pltpu.get_barrier_semaphore()
        pl.semaphore_signal(
            barrier, inc=1, device_id=nbr, device_id_type=pl.DeviceIdType.MESH
        )
        pl.semaphore_wait(barrier, 1)

        def wload(cb):
            return pltpu.make_async_copy(
                W_ref.at[:, pl.ds(cb * CB, CB)], wstage, wload_sem
            )

        def chunk_rdma(cb):
            return pltpu.make_async_remote_copy(
                src_ref=sbuf.at[cb],
                dst_ref=rbuf.at[cb],
                send_sem=send_sems.at[cb],
                recv_sem=recv_sems.at[cb],
                device_id=nbr,
                device_id_type=pl.DeviceIdType.MESH,
            )

        mrun[...] = jnp.full((T, 1), -jnp.inf, jnp.float32)
        srun[...] = jnp.zeros((T, 1), jnp.float32)
        wload(0).start()

        def phase1(cb, carry):
            wload(cb).wait()
            wbuf[...] = wstage[...].astype(jnp.bfloat16)

            @pl.when(cb + 1 < NCB)
            def _():
                wload(cb + 1).start()

            logits = jnp.dot(
                x_ref[...], wbuf[...], preferred_element_type=jnp.float32
            )
            m_cb = jnp.max(logits, axis=1, keepdims=True)
            e = jnp.exp(logits - m_cb)
            s_cb = jnp.sum(e, axis=1, keepdims=True)
            lane = lax.broadcasted_iota(jnp.int32, (T, 32), 1)
            stats_mine[...] = jnp.where(lane == cb, m_cb, stats_mine[...])
            m_new = jnp.maximum(mrun[...], m_cb)
            srun[...] = srun[...] * jnp.exp(mrun[...] - m_new) + s_cb * jnp.exp(
                m_cb - m_new
            )
            mrun[...] = m_new
            ebuf[...] = e.astype(jnp.bfloat16)
            stage = pltpu.make_async_copy(ebuf, sbuf.at[cb], estage_sem)
            stage.start()
            stage.wait()
            chunk_rdma(cb).start()
            return carry

        lax.fori_loop(0, NCB, phase1, 0)

        stats_mine[:, ML : ML + 1] = mrun[...]
        stats_mine[:, SL : SL + 1] = srun[...]
        st = pltpu.make_async_remote_copy(
            src_ref=stats_mine,
            dst_ref=stats_theirs,
            send_sem=stat_send_sem,
            recv_sem=stat_recv_sem,
            device_id=nbr,
            device_id_type=pl.DeviceIdType.MESH,
        )
        st.start()
        st.wait()

        m_loc = stats_mine[:, ML : ML + 1]
        s_loc = stats_mine[:, SL : SL + 1]
        m_rem = stats_theirs[:, ML : ML + 1]
        s_rem = stats_theirs[:, SL : SL + 1]
        mm = jnp.maximum(m_loc, m_rem)
        ss = s_loc * jnp.exp(m_loc - mm) + s_rem * jnp.exp(m_rem - mm)
        scales[:, 0:NCB] = jnp.exp(stats_mine[:, 0:NCB] - mm) / ss
        scales[:, 16 : 16 + NCB] = jnp.exp(stats_theirs[:, 0:NCB] - mm) / ss

        def store_dma(slot, cb, col0):
            return pltpu.make_async_copy(
                obuf.at[slot],
                out_ref.at[:, pl.ds(col0 + cb * CB, CB)],
                ostore_sems.at[slot],
            )

        def emit_loop(src_hbm, lane0, col0, wait_recv_first):
            def emit(cb, carry):
                slot = lax.rem(cb, 2)

                @pl.when(cb >= 2)
                def _():
                    store_dma(slot, cb - 2, col0).wait()

                if wait_recv_first:
                    chunk_rdma(cb).wait_recv()
                ld = pltpu.make_async_copy(
                    src_hbm.at[cb], rconv.at[slot], rload_sems.at[slot]
                )
                ld.start()
                ld.wait()
                lane = lax.broadcasted_iota(jnp.int32, (T, 32), 1)
                svec = jnp.sum(
                    jnp.where(lane == lane0 + cb, scales[...], 0.0),
                    axis=1,
                    keepdims=True,
                )
                obuf[slot] = (
                    rconv[slot].astype(jnp.float32) * svec
                ).astype(jnp.bfloat16)
                store_dma(slot, cb, col0).start()
                return carry

            lax.fori_loop(0, NCB, emit, 0)
            for last in (NCB - 2, NCB - 1):
                store_dma(last % 2, last, col0).wait()

        emit_loop(sbuf, 0, my_x * V_HALF, False)
        emit_loop(rbuf, 16, (1 - my_x) * V_HALF, True)

        def waitsend(cb, carry):
            chunk_rdma(cb).wait_send()
            return carry

        lax.fori_loop(0, NCB, waitsend, 0)

    out = pl.pallas_call(
        body,
        out_shape=[
            jax.ShapeDtypeStruct((T, V), jnp.bfloat16),
            jax.ShapeDtypeStruct((NCB, T, CB), jnp.bfloat16),
            jax.ShapeDtypeStruct((NCB, T, CB), jnp.bfloat16),
        ],
        in_specs=[
            pl.BlockSpec(memory_space=pltpu.MemorySpace.VMEM),
            pl.BlockSpec(memory_space=pltpu.MemorySpace.HBM),
        ],
        out_specs=[
            pl.BlockSpec(memory_space=pltpu.MemorySpace.HBM),
            pl.BlockSpec(memory_space=pltpu.MemorySpace.HBM),
            pl.BlockSpec(memory_space=pltpu.MemorySpace.HBM),
        ],
        scratch_shapes=[
            pltpu.VMEM((D, CB), jnp.float32),
            pltpu.VMEM((D, CB), jnp.bfloat16),
            pltpu.VMEM((T, CB), jnp.bfloat16),
            pltpu.VMEM((2, T, CB), jnp.bfloat16),
            pltpu.VMEM((2, T, CB), jnp.bfloat16),
            pltpu.VMEM((T, 32), jnp.float32),
            pltpu.VMEM((T, 32), jnp.float32),
            pltpu.VMEM((T, 32), jnp.float32),
            pltpu.VMEM((T, 1), jnp.float32),
            pltpu.VMEM((T, 1), jnp.float32),
            pltpu.SemaphoreType.DMA,
            pltpu.SemaphoreType.DMA,
            pltpu.SemaphoreType.DMA((2,)),
            pltpu.SemaphoreType.DMA((2,)),
            pltpu.SemaphoreType.DMA,
            pltpu.SemaphoreType.DMA,
            pltpu.SemaphoreType.DMA((NCB,)),
            pltpu.SemaphoreType.DMA((NCB,)),
        ],
        compiler_params=pltpu.CompilerParams(
            collective_id=0, vmem_limit_bytes=56 * 1024 * 1024
        ),
    )(x_bf16, W)
    return out[0]


def kernel(x, W):
    return _fused(x.astype(jnp.bfloat16), W)


# device time: 529507 ns/iter; 4.6356x vs baseline; 1.0394x over previous
import jax
import jax.numpy as jnp
from jax import lax
from jax.experimental import pallas as pl
from jax.experimental.pallas import tpu as pltpu

T = 2048
D = 4096
V_HALF = 8192
V = 2 * V_HALF
CB = 512
NCB = V_HALF // CB
assert NCB <= 16

ML = 16
SL = 17


def _fused(x_bf16, W):
    def body(
        x_ref,
        W_ref,
        out_ref,
        sbuf,
        rbuf,
        wstage,
        wbuf,
        ebuf,
        rconv,
        obuf,
        stats_mine,
        stats_theirs,
        scales,
        mrun, srun,
        wload_sem, estage_sem, rload_sems, ostore_sems,
        stat_send_sem, stat_recv_sem,
        send_sems, recv_sems,
    ):
        my_x = lax.axis_index("x")
        my_y = lax.axis_index("y")
        nbr = (1 - my_x, my_y)

        barrier = pltpu.get_barrier_semaphore()
        pl.semaphore_signal(
            barrier, inc=1, device_id=nbr, device_id_type=pl.DeviceIdType.MESH
        )
        pl.semaphore_wait(barrier, 1)

        def wload(cb):
            return pltpu.make_async_copy(
                W_ref.at[:, pl.ds(cb * CB, CB)], wstage, wload_sem
            )

        def chunk_rdma(cb):
            return pltpu.make_async_remote_copy(
                src_ref=sbuf.at[cb],
                dst_ref=rbuf.at[cb],
                send_sem=send_sems.at[cb],
                recv_sem=recv_sems.at[cb],
                device_id=nbr,
                device_id_type=pl.DeviceIdType.MESH,
            )

        mrun[...] = jnp.full((T, 1), -jnp.inf, jnp.float32)
        srun[...] = jnp.zeros((T, 1), jnp.float32)
        wload(0).start()

        def phase1(cb, carry):
            wload(cb).wait()
            wbuf[...] = wstage[...].astype(jnp.bfloat16)

            @pl.when(cb + 1 < NCB)
            def _():
                wload(cb + 1).start()

            logits = jnp.dot(
                x_ref[...], wbuf[...], preferred_element_type=jnp.float32
            )
            m_cb = jnp.max(logits, axis=1, keepdims=True)
            e = jnp.exp(logits - m_cb)
            s_cb = jnp.sum(e, axis=1, keepdims=True)
            lane = lax.broadcasted_iota(jnp.int32, (T, 32), 1)
            stats_mine[...] = jnp.where(lane == cb, m_cb, stats_mine[...])
            m_new = jnp.maximum(mrun[...], m_cb)
            srun[...] = srun[...] * jnp.exp(mrun[...] - m_new) + s_cb * jnp.exp(
                m_cb - m_new
            )
            mrun[...] = m_new
            ebuf[...] = e.astype(jnp.bfloat16)
            stage = pltpu.make_async_copy(ebuf, sbuf.at[cb], estage_sem)
            stage.start()
            stage.wait()
            chunk_rdma(cb).start()
            return carry

        lax.fori_loop(0, NCB, phase1, 0)

        stats_mine[:, ML : ML + 1] = mrun[...]
        stats_mine[:, SL : SL + 1] = srun[...]
        st = pltpu.make_async_remote_copy(
            src_ref=stats_mine,
            dst_ref=stats_theirs,
            send_sem=stat_send_sem,
            recv_sem=stat_recv_sem,
            device_id=nbr,
            device_id_type=pl.DeviceIdType.MESH,
        )
        st.start()
        st.wait()

        m_loc = stats_mine[:, ML : ML + 1]
        s_loc = stats_mine[:, SL : SL + 1]
        m_rem = stats_theirs[:, ML : ML + 1]
        s_rem = stats_theirs[:, SL : SL + 1]
        mm = jnp.maximum(m_loc, m_rem)
        ss = s_loc * jnp.exp(m_loc - mm) + s_rem * jnp.exp(m_rem - mm)
        scales[:, 0:NCB] = jnp.exp(stats_mine[:, 0:NCB] - mm) / ss
        scales[:, 16 : 16 + NCB] = jnp.exp(stats_theirs[:, 0:NCB] - mm) / ss

        def store_dma(slot, cb, col0):
            return pltpu.make_async_copy(
                obuf.at[slot],
                out_ref.at[:, pl.ds(col0 + cb * CB, CB)],
                ostore_sems.at[slot],
            )

        def emit_loop(src_hbm, lane0, col0, wait_recv_first):
            def eload(cb):
                return pltpu.make_async_copy(
                    src_hbm.at[cb],
                    rconv.at[lax.rem(cb, 2)],
                    rload_sems.at[lax.rem(cb, 2)],
                )

            if wait_recv_first:
                chunk_rdma(0).wait_recv()
            eload(0).start()

            def emit(cb, carry):
                slot = lax.rem(cb, 2)

                @pl.when(cb >= 2)
                def _():
                    store_dma(slot, cb - 2, col0).wait()

                eload(cb).wait()

                @pl.when(cb + 1 < NCB)
                def _():
                    if wait_recv_first:
                        chunk_rdma(cb + 1).wait_recv()
                    eload(cb + 1).start()

                lane = lax.broadcasted_iota(jnp.int32, (T, 32), 1)
                svec = jnp.sum(
                    jnp.where(lane == lane0 + cb, scales[...], 0.0),
                    axis=1,
                    keepdims=True,
                )
                obuf[slot] = (
                    rconv[slot].astype(jnp.float32) * svec
                ).astype(jnp.bfloat16)
                store_dma(slot, cb, col0).start()
                return carry

            lax.fori_loop(0, NCB, emit, 0)
            for last in (NCB - 2, NCB - 1):
                store_dma(last % 2, last, col0).wait()

        emit_loop(sbuf, 0, my_x * V_HALF, False)
        emit_loop(rbuf, 16, (1 - my_x) * V_HALF, True)

        def waitsend(cb, carry):
            chunk_rdma(cb).wait_send()
            return carry

        lax.fori_loop(0, NCB, waitsend, 0)

    out = pl.pallas_call(
        body,
        out_shape=[
            jax.ShapeDtypeStruct((T, V), jnp.bfloat16),
            jax.ShapeDtypeStruct((NCB, T, CB), jnp.bfloat16),
            jax.ShapeDtypeStruct((NCB, T, CB), jnp.bfloat16),
        ],
        in_specs=[
            pl.BlockSpec(memory_space=pltpu.MemorySpace.VMEM),
            pl.BlockSpec(memory_space=pltpu.MemorySpace.HBM),
        ],
        out_specs=[
            pl.BlockSpec(memory_space=pltpu.MemorySpace.HBM),
            pl.BlockSpec(memory_space=pltpu.MemorySpace.HBM),
            pl.BlockSpec(memory_space=pltpu.MemorySpace.HBM),
        ],
        scratch_shapes=[
            pltpu.VMEM((D, CB), jnp.float32),
            pltpu.VMEM((D, CB), jnp.bfloat16),
            pltpu.VMEM((T, CB), jnp.bfloat16),
            pltpu.VMEM((2, T, CB), jnp.bfloat16),
            pltpu.VMEM((2, T, CB), jnp.bfloat16),
            pltpu.VMEM((T, 32), jnp.float32),
            pltpu.VMEM((T, 32), jnp.float32),
            pltpu.VMEM((T, 32), jnp.float32),
            pltpu.VMEM((T, 1), jnp.float32),
            pltpu.VMEM((T, 1), jnp.float32),
            pltpu.SemaphoreType.DMA,
            pltpu.SemaphoreType.DMA,
            pltpu.SemaphoreType.DMA((2,)),
            pltpu.SemaphoreType.DMA((2,)),
            pltpu.SemaphoreType.DMA,
            pltpu.SemaphoreType.DMA,
            pltpu.SemaphoreType.DMA((NCB,)),
            pltpu.SemaphoreType.DMA((NCB,)),
        ],
        compiler_params=pltpu.CompilerParams(
            collective_id=0, vmem_limit_bytes=56 * 1024 * 1024
        ),
    )(x_bf16, W)
    return out[0]


def kernel(x, W):
    return _fused(x.astype(jnp.bfloat16), W)
